# baseline (speedup 1.0000x reference)
import functools

import jax
import jax.numpy as jnp
from jax import lax
from jax.experimental import pallas as pl
from jax.experimental.pallas import tpu as pltpu
from jax.experimental.pallas import tpu_sc as plsc

N_CODES = 8192
DIM = 32
TOK_TILE = 4096

SC_CORES = 2
SC_SUBCORES = 16
SC_WORKERS = SC_CORES * SC_SUBCORES
GATHER_CHUNK = 128


ROWS = 64
LCH = 128


GROUP = 256


def _argmin_body(x_ref, e_ref, xsqb_ref, esq_ref, ids_ref, idx_ref):
    e = e_ref[...]
    esq = esq_ref[...]
    ids = ids_ref[...]
    nkc = N_CODES // LCH
    out = []
    for g in range(TOK_TILE // GROUP):
        xg = x_ref[g * GROUP:(g + 1) * GROUP, :]
        simg = jnp.dot(xg, e, preferred_element_type=jnp.float32)
        for r0 in range(GROUP // ROWS):
            r = g * (GROUP // ROWS) + r0
            xb = xsqb_ref[r * ROWS:(r + 1) * ROWS, :]
            srow = simg[r0 * ROWS:(r0 + 1) * ROWS, :]
            run_min = (xb + esq[:, :LCH]) - 2.0 * srow[:, :LCH]
            run_idx = jnp.broadcast_to(ids[:, :LCH], (ROWS, LCH))
            for k in range(1, nkc):
                dk = (xb + esq[:, k * LCH:(k + 1) * LCH]) - 2.0 * srow[:, k * LCH:(k + 1) * LCH]
                cond = dk < run_min
                run_min = jnp.where(cond, dk, run_min)
                run_idx = jnp.where(cond, ids[:, k * LCH:(k + 1) * LCH], run_idx)
            dmin = jnp.min(run_min, axis=1, keepdims=True)
            idxr = jnp.min(jnp.where(run_min == dmin, run_idx, float(N_CODES)), axis=1)
            out.append(idxr)
    idx_ref[0, 0, :] = jnp.concatenate(out).astype(jnp.int32)


def _nearest_code_indices(x2, embedding, xsq, esq, ids):
    n_tiles = x2.shape[0] // TOK_TILE
    idx3 = pl.pallas_call(
        _argmin_body,
        grid=(n_tiles,),
        in_specs=[
            pl.BlockSpec((TOK_TILE, DIM), lambda i: (i, 0)),
            pl.BlockSpec((DIM, N_CODES), lambda i: (0, 0)),
            pl.BlockSpec((TOK_TILE, LCH), lambda i: (i, 0)),
            pl.BlockSpec((1, N_CODES), lambda i: (0, 0)),
            pl.BlockSpec((1, N_CODES), lambda i: (0, 0)),
        ],
        out_specs=pl.BlockSpec((1, 1, TOK_TILE), lambda i: (i, 0, 0)),
        out_shape=jax.ShapeDtypeStruct((n_tiles, 1, TOK_TILE), jnp.int32),
    )(x2, embedding, xsq, esq, ids)
    return idx3.reshape(-1)


def _gather_body(table_hbm, idx_hbm, out_hbm, idx_v, rows_v, sem):
    wid = lax.axis_index("s") * SC_CORES + lax.axis_index("c")
    chunks_per_w = idx_hbm.shape[0] // SC_WORKERS
    base_chunk = wid * chunks_per_w
    pltpu.sync_copy(idx_hbm.at[pl.ds(base_chunk, chunks_per_w)], idx_v)
    for j in range(chunks_per_w):
        pltpu.async_copy(
            table_hbm.at[idx_v.at[j]],
            rows_v.at[pl.ds(j * GATHER_CHUNK, GATHER_CHUNK)],
            sem,
        ).wait()
    rows_per_w = chunks_per_w * GATHER_CHUNK
    pltpu.sync_copy(rows_v, out_hbm.at[pl.ds(wid * rows_per_w, rows_per_w)])


def _gather_rows(table, idx):
    n_rows = idx.shape[0]
    chunks_per_w = n_rows // (SC_WORKERS * GATHER_CHUNK)
    rows_per_w = chunks_per_w * GATHER_CHUNK
    idx2 = idx.reshape(n_rows // GATHER_CHUNK, GATHER_CHUNK)
    mesh = plsc.VectorSubcoreMesh(core_axis_name="c", subcore_axis_name="s")
    fn = functools.partial(
        pl.kernel,
        mesh=mesh,
        out_type=jax.ShapeDtypeStruct((n_rows, DIM), jnp.float32),
        scratch_types=[
            pltpu.VMEM((chunks_per_w, GATHER_CHUNK), jnp.int32),
            pltpu.VMEM((rows_per_w, DIM), jnp.float32),
            pltpu.SemaphoreType.DMA,
        ],
        compiler_params=pltpu.CompilerParams(use_tc_tiling_on_sc=False),
    )(_gather_body)
    return fn(table, idx2)


def kernel(x, embedding):
    flat = x.reshape(-1, DIM)
    xsq = jnp.sum(flat**2, axis=1, keepdims=True)
    xsqb = jnp.broadcast_to(xsq, (flat.shape[0], LCH))
    esq = jnp.sum(embedding**2, axis=0, keepdims=True)
    ids = lax.broadcasted_iota(jnp.float32, (1, N_CODES), 1)
    idx = _nearest_code_indices(flat, embedding, xsqb, esq, ids)
    quantized = _gather_rows(embedding.T, idx)
    return quantized.reshape(x.shape)


# speedup vs baseline: 1.0087x; 1.0087x over previous
import functools

import jax
import jax.numpy as jnp
from jax import lax
from jax.experimental import pallas as pl
from jax.experimental.pallas import tpu as pltpu
from jax.experimental.pallas import tpu_sc as plsc

N_CODES = 8192
DIM = 32
TOK_TILE = 2048

SC_CORES = 2
SC_SUBCORES = 16
SC_WORKERS = SC_CORES * SC_SUBCORES
GATHER_CHUNK = 128


ROWS = 64
LCH = 128


GROUP = 256


def _argmin_body(x_ref, e_ref, xsqb_ref, esq_ref, ids_ref, idx_ref):
    e = e_ref[...]
    esq = esq_ref[...]
    ids = ids_ref[...]
    nkc = N_CODES // LCH
    out = []
    for g in range(TOK_TILE // GROUP):
        xg = x_ref[g * GROUP:(g + 1) * GROUP, :]
        simg = jnp.dot(xg, e, preferred_element_type=jnp.float32)
        for r0 in range(GROUP // ROWS):
            r = g * (GROUP // ROWS) + r0
            xb = xsqb_ref[r * ROWS:(r + 1) * ROWS, :]
            srow = simg[r0 * ROWS:(r0 + 1) * ROWS, :]
            run_min = (xb + esq[:, :LCH]) - 2.0 * srow[:, :LCH]
            run_idx = jnp.broadcast_to(ids[:, :LCH], (ROWS, LCH))
            for k in range(1, nkc):
                dk = (xb + esq[:, k * LCH:(k + 1) * LCH]) - 2.0 * srow[:, k * LCH:(k + 1) * LCH]
                cond = dk < run_min
                run_min = jnp.where(cond, dk, run_min)
                run_idx = jnp.where(cond, ids[:, k * LCH:(k + 1) * LCH], run_idx)
            dmin = jnp.min(run_min, axis=1, keepdims=True)
            idxr = jnp.min(jnp.where(run_min == dmin, run_idx, float(N_CODES)), axis=1)
            out.append(idxr)
    idx_ref[0, 0, :] = jnp.concatenate(out).astype(jnp.int32)


def _nearest_code_indices(x2, embedding, xsq, esq, ids):
    n_tiles = x2.shape[0] // TOK_TILE
    idx3 = pl.pallas_call(
        _argmin_body,
        grid=(n_tiles,),
        in_specs=[
            pl.BlockSpec((TOK_TILE, DIM), lambda i: (i, 0)),
            pl.BlockSpec((DIM, N_CODES), lambda i: (0, 0)),
            pl.BlockSpec((TOK_TILE, LCH), lambda i: (i, 0)),
            pl.BlockSpec((1, N_CODES), lambda i: (0, 0)),
            pl.BlockSpec((1, N_CODES), lambda i: (0, 0)),
        ],
        out_specs=pl.BlockSpec((1, 1, TOK_TILE), lambda i: (i, 0, 0)),
        out_shape=jax.ShapeDtypeStruct((n_tiles, 1, TOK_TILE), jnp.int32),
    )(x2, embedding, xsq, esq, ids)
    return idx3.reshape(-1)


def _gather_body(table_hbm, idx_hbm, out_hbm, idx_v, rows_v, sem):
    wid = lax.axis_index("s") * SC_CORES + lax.axis_index("c")
    chunks_per_w = idx_hbm.shape[0] // SC_WORKERS
    base_chunk = wid * chunks_per_w
    pltpu.sync_copy(idx_hbm.at[pl.ds(base_chunk, chunks_per_w)], idx_v)
    copies = [
        pltpu.async_copy(
            table_hbm.at[idx_v.at[j]],
            rows_v.at[pl.ds(j * GATHER_CHUNK, GATHER_CHUNK)],
            sem,
        )
        for j in range(chunks_per_w)
    ]
    for c in copies:
        c.wait()
    rows_per_w = chunks_per_w * GATHER_CHUNK
    pltpu.sync_copy(rows_v, out_hbm.at[pl.ds(wid * rows_per_w, rows_per_w)])


def _gather_rows(table, idx):
    n_rows = idx.shape[0]
    chunks_per_w = n_rows // (SC_WORKERS * GATHER_CHUNK)
    rows_per_w = chunks_per_w * GATHER_CHUNK
    idx2 = idx.reshape(n_rows // GATHER_CHUNK, GATHER_CHUNK)
    mesh = plsc.VectorSubcoreMesh(core_axis_name="c", subcore_axis_name="s")
    fn = functools.partial(
        pl.kernel,
        mesh=mesh,
        out_type=jax.ShapeDtypeStruct((n_rows, DIM), jnp.float32),
        scratch_types=[
            pltpu.VMEM((chunks_per_w, GATHER_CHUNK), jnp.int32),
            pltpu.VMEM((rows_per_w, DIM), jnp.float32),
            pltpu.SemaphoreType.DMA,
        ],
        compiler_params=pltpu.CompilerParams(use_tc_tiling_on_sc=False),
    )(_gather_body)
    return fn(table, idx2)


def kernel(x, embedding):
    flat = x.reshape(-1, DIM)
    xsq = jnp.sum(flat**2, axis=1, keepdims=True)
    xsqb = jnp.broadcast_to(xsq, (flat.shape[0], LCH))
    esq = jnp.sum(embedding**2, axis=0, keepdims=True)
    ids = lax.broadcasted_iota(jnp.float32, (1, N_CODES), 1)
    idx = _nearest_code_indices(flat, embedding, xsqb, esq, ids)
    quantized = _gather_rows(embedding.T, idx)
    return quantized.reshape(x.shape)
